# baseline (speedup 1.0000x reference)
import jax
import jax.numpy as jnp
import numpy as np
from jax import lax
from jax.experimental import pallas as pl
from jax.experimental.pallas import tpu as pltpu
from jax.experimental.pallas import tpu_sc as plsc

N = 10000
D = 128
H = D // 2
E = 320000

NC = 2
NS = 16

B = 128
E_PAD = 327680
PAD = E_PAD - E
R = E_PAD // B
NB = R // NS
NIT = NB // 4
N_PAD = 10112
ROWS_PER_TILE = N_PAD // NS


def _make_agg(with_counts):
  mesh = plsc.VectorSubcoreMesh(core_axis_name="c", subcore_axis_name="s")
  out_type = [jax.ShapeDtypeStruct((N_PAD, D), jnp.float32)]
  scratch = [
      pltpu.VMEM((NB, B), jnp.int32),
      pltpu.VMEM((NB, B), jnp.int32),
      pltpu.VMEM((B, H), jnp.float32),
      pltpu.VMEM((B, H), jnp.float32),
      pltpu.VMEM((B, H), jnp.float32),
      pltpu.VMEM((B, H), jnp.float32),
      pltpu.VMEM_SHARED((N_PAD, H), jnp.float32),
      pltpu.SemaphoreType.DMA,
      pltpu.SemaphoreType.DMA,
      pltpu.SemaphoreType.DMA,
      pltpu.SemaphoreType.DMA,
      pltpu.SemaphoreType.DMA,
      pltpu.SemaphoreType.DMA,
      pltpu.SemaphoreType.DMA,
      pltpu.SemaphoreType.DMA,
  ]
  if with_counts:
    out_type.append(jax.ShapeDtypeStruct((N_PAD, D), jnp.float32))
    scratch += [
        pltpu.VMEM((B, 16), jnp.float32),
        pltpu.VMEM((B, 16), jnp.float32),
        pltpu.VMEM_SHARED((N_PAD, 16), jnp.float32),
    ]

  def body(y2, src2, dst2, *refs):
    if with_counts:
      (p_out, c_out, sidx, didx, buf_a, buf_b, buf_c, buf_d, acc,
       sem_a, sem_b, sem_c, sem_d, ss_a, ss_b, ss_c, ss_d,
       ones_v, zer16, cnt) = refs
    else:
      (p_out, sidx, didx, buf_a, buf_b, buf_c, buf_d, acc,
       sem_a, sem_b, sem_c, sem_d, ss_a, ss_b, ss_c, ss_d) = refs
    ci = lax.axis_index("c")
    si = lax.axis_index("s")

    def zf(i, _):
      buf_a[i // (H // 16), pl.ds((i % (H // 16)) * 16, 16)] = (
          jnp.zeros((16,), jnp.float32))
      return 0
    lax.fori_loop(0, B * H // 16, zf, 0)
    r0 = si * ROWS_PER_TILE
    for k in range(ROWS_PER_TILE // B):
      pltpu.sync_copy(buf_a, acc.at[pl.ds(r0 + k * B, B)])
    rem = ROWS_PER_TILE % B
    if rem:
      pltpu.sync_copy(buf_a.at[pl.ds(0, rem)],
                      acc.at[pl.ds(r0 + (ROWS_PER_TILE // B) * B, rem)])

    if with_counts:
      def of(i, _):
        ones_v[i, pl.ds(0, 16)] = jnp.ones((16,), jnp.float32)
        zer16[i, pl.ds(0, 16)] = jnp.zeros((16,), jnp.float32)
        return 0
      lax.fori_loop(0, B, of, 0)
      for k in range(ROWS_PER_TILE // B):
        pltpu.sync_copy(zer16, cnt.at[pl.ds(r0 + k * B, B)])
      if rem:
        pltpu.sync_copy(zer16.at[pl.ds(0, rem)],
                        cnt.at[pl.ds(r0 + (ROWS_PER_TILE // B) * B, rem)])

    plsc.subcore_barrier()

    er0 = si * NB
    pltpu.sync_copy(src2.at[pl.ds(er0, NB)], sidx)
    pltpu.sync_copy(dst2.at[pl.ds(er0, NB)], didx)

    def dbl(i, _):
      r = i // (B // 16)
      c0 = (i % (B // 16)) * 16
      v = sidx[r, pl.ds(c0, 16)]
      sidx[r, pl.ds(c0, 16)] = v + v + ci
      return 0
    lax.fori_loop(0, NB * (B // 16), dbl, 0)

    table = y2

    def start_gather(b, buf, sem):
      pltpu.make_async_copy(table.at[sidx.at[b]], buf, sem).start()

    def wait_gather(b, buf, sem):
      pltpu.make_async_copy(table.at[sidx.at[b]], buf, sem).wait()

    def mk_scatter(b, buf, ssem):
      return pltpu.make_async_copy(buf, acc.at[didx.at[b]], ssem)

    slots = ((buf_a, sem_a, 0, ss_a), (buf_b, sem_b, 1, ss_b),
             (buf_c, sem_c, 0, ss_c), (buf_d, sem_d, 1, ss_d))
    NBT = NB

    start_gather(0, buf_a, sem_a)
    start_gather(1, buf_b, sem_b)

    def it(i, _):
      for j, (buf, gsem, parity, ssem) in enumerate(slots):
        b = 4 * i + j
        wait_gather(b, buf, gsem)
        mk_scatter(b, buf, ssem).start(add=True)
        if with_counts:
          @pl.when(ci == parity)
          def _():
            pltpu.sync_copy(ones_v, cnt.at[didx.at[b]], add=True)
        rbuf, rgsem, _, rssem = slots[(j + 2) % 4]

        @pl.when(b + 2 < NBT)
        def _():
          @pl.when(b >= 2)
          def _():
            mk_scatter(b - 2, rbuf, rssem).wait()
          start_gather(b + 2, rbuf, rgsem)
      return 0

    lax.fori_loop(0, NIT, it, 0)
    for j in range(4):
      mk_scatter(NBT - 4 + j, slots[j][0], slots[j][3]).wait()
    plsc.subcore_barrier()

    pltpu.sync_copy(acc.at[pl.ds(r0, ROWS_PER_TILE)],
                    p_out.at[pl.ds(r0, ROWS_PER_TILE), pl.ds(ci * H, H)])
    if with_counts:
      pltpu.sync_copy(cnt.at[pl.ds(r0, ROWS_PER_TILE)],
                      c_out.at[pl.ds(r0, ROWS_PER_TILE), pl.ds(ci * 16, 16)])

  return pl.kernel(
      body, out_type=out_type, mesh=mesh, scratch_types=scratch,
      compiler_params=pltpu.CompilerParams(use_tc_tiling_on_sc=False))


_agg_with_counts = _make_agg(True)
_agg_plain = _make_agg(False)

BM = 2000


def _pre(x, W_l):
  def body(x_ref, w_ref, o_ref):
    o_ref[...] = lax.dot_general(x_ref[...], w_ref[...],
                                 (((1,), (1,)), ((), ())),
                                 preferred_element_type=jnp.float32)
  return pl.pallas_call(
      body,
      grid=(N // BM,),
      in_specs=[pl.BlockSpec((BM, D), lambda i: (i, 0)),
                pl.BlockSpec((D, D), lambda i: (0, 0))],
      out_specs=pl.BlockSpec((BM, D), lambda i: (i, 0)),
      out_shape=jax.ShapeDtypeStruct((N, D), jnp.float32),
  )(x, W_l)


def _unpack_mean(p_ref, c_ref):
  cnt = c_ref[:, 0:1] + c_ref[:, 16:17]
  return p_ref[...] * (1.0 / jnp.maximum(cnt, 1.0))


def _post(p, cnts, b, x, W_r, relu):
  def body(p_ref, c_ref, b_ref, x_ref, w_ref, o_ref):
    r = _unpack_mean(p_ref, c_ref) + b_ref[...] + lax.dot_general(
        x_ref[...], w_ref[...], (((1,), (1,)), ((), ())),
        preferred_element_type=jnp.float32)
    o_ref[...] = jnp.maximum(r, 0.0) if relu else r
  return pl.pallas_call(
      body,
      grid=(N // BM,),
      in_specs=[
          pl.BlockSpec((BM, D), lambda i: (i, 0)),
          pl.BlockSpec((BM, D), lambda i: (i, 0)),
          pl.BlockSpec((1, D), lambda i: (0, 0)),
          pl.BlockSpec((BM, D), lambda i: (i, 0)),
          pl.BlockSpec((D, D), lambda i: (0, 0)),
      ],
      out_specs=pl.BlockSpec((BM, D), lambda i: (i, 0)),
      out_shape=jax.ShapeDtypeStruct((N, D), jnp.float32),
  )(p, cnts, b, x, W_r)


def _post_pre(p, cnts, b, x, W_r, W_ln):
  def body(p_ref, c_ref, b_ref, x_ref, w_ref, wl_ref, h_ref, y_ref):
    r = _unpack_mean(p_ref, c_ref) + b_ref[...] + lax.dot_general(
        x_ref[...], w_ref[...], (((1,), (1,)), ((), ())),
        preferred_element_type=jnp.float32)
    h = jnp.maximum(r, 0.0)
    h_ref[...] = h
    y_ref[...] = lax.dot_general(h, wl_ref[...], (((1,), (1,)), ((), ())),
                                 preferred_element_type=jnp.float32)
  return pl.pallas_call(
      body,
      grid=(N // BM,),
      in_specs=[
          pl.BlockSpec((BM, D), lambda i: (i, 0)),
          pl.BlockSpec((BM, D), lambda i: (i, 0)),
          pl.BlockSpec((1, D), lambda i: (0, 0)),
          pl.BlockSpec((BM, D), lambda i: (i, 0)),
          pl.BlockSpec((D, D), lambda i: (0, 0)),
          pl.BlockSpec((D, D), lambda i: (0, 0)),
      ],
      out_specs=[pl.BlockSpec((BM, D), lambda i: (i, 0)),
                 pl.BlockSpec((BM, D), lambda i: (i, 0))],
      out_shape=[jax.ShapeDtypeStruct((N, D), jnp.float32),
                 jax.ShapeDtypeStruct((N, D), jnp.float32)],
  )(p, cnts, b, x, W_r, W_ln)


def kernel(x, edge_index, W1_l, b1, W1_r, W2_l, b2, W2_r):
  ei = edge_index.astype(jnp.int32)
  pad_i = np.arange(PAD, dtype=np.int32)
  pad_src = jnp.asarray(((pad_i * 37) % N).reshape(PAD // B, B))
  pad_dst = jnp.asarray((N + pad_i % (N_PAD - N)).reshape(PAD // B, B))
  src2 = jnp.concatenate([ei[0].reshape(E // B, B), pad_src], axis=0)
  dst2 = jnp.concatenate([ei[1].reshape(E // B, B), pad_dst], axis=0)
  src2, dst2 = lax.optimization_barrier((src2, dst2))
  b1r = b1.reshape(1, D)
  b2r = b2.reshape(1, D)

  y1 = jnp.reshape(_pre(x, W1_l), (2 * N, H))
  p1, cnts = _agg_with_counts(y1, src2, dst2)
  h, y2 = _post_pre(p1, cnts, b1r, x, W1_r, W2_l)
  (p2,) = _agg_plain(jnp.reshape(y2, (2 * N, H)), src2, dst2)
  out = _post(p2, cnts, b2r, h, W2_r, relu=False)
  return out


# speedup vs baseline: 1.2346x; 1.2346x over previous
import jax
import jax.numpy as jnp
import numpy as np
from jax import lax
from jax.experimental import pallas as pl
from jax.experimental.pallas import tpu as pltpu
from jax.experimental.pallas import tpu_sc as plsc

N = 10000
D = 128
H = D // 2
E = 320000

NC = 2
NS = 16

B = 128
E_PAD = 327680
PAD = E_PAD - E
R = E_PAD // B
NB = R // NS
NIT = NB // 4
N_PAD = 10112
ROWS_PER_TILE = N_PAD // NS


def _make_agg(with_counts):
  mesh = plsc.VectorSubcoreMesh(core_axis_name="c", subcore_axis_name="s")
  out_type = [jax.ShapeDtypeStruct((N_PAD, D), jnp.float32)]
  scratch = [
      pltpu.VMEM((NB, B), jnp.int32),
      pltpu.VMEM((NB, B), jnp.int32),
      pltpu.VMEM((B, H), jnp.float32),
      pltpu.VMEM((B, H), jnp.float32),
      pltpu.VMEM((B, H), jnp.float32),
      pltpu.VMEM((B, H), jnp.float32),
      pltpu.VMEM_SHARED((N_PAD, H), jnp.float32),
      pltpu.SemaphoreType.DMA,
      pltpu.SemaphoreType.DMA,
      pltpu.SemaphoreType.DMA,
      pltpu.SemaphoreType.DMA,
  ]
  if with_counts:
    out_type.append(jax.ShapeDtypeStruct((N_PAD, D), jnp.float32))
    scratch += [
        pltpu.VMEM((B, 16), jnp.float32),
        pltpu.VMEM((B, 16), jnp.float32),
        pltpu.VMEM_SHARED((N_PAD, 16), jnp.float32),
    ]

  def body(y2, src2, dst2, *refs):
    if with_counts:
      (p_out, c_out, sidx, didx, buf_a, buf_b, buf_c, buf_d, acc,
       sem_a, sem_b, sem_c, sem_d, ones_v, zer16, cnt) = refs
    else:
      (p_out, sidx, didx, buf_a, buf_b, buf_c, buf_d, acc,
       sem_a, sem_b, sem_c, sem_d) = refs
    ci = lax.axis_index("c")
    si = lax.axis_index("s")

    er0 = si * NB
    pltpu.sync_copy(src2.at[pl.ds(er0, NB)], sidx)
    pltpu.sync_copy(dst2.at[pl.ds(er0, NB)], didx)

    def dbl(i, _):
      r = i // (B // 16)
      c0 = (i % (B // 16)) * 16
      v = sidx[r, pl.ds(c0, 16)]
      sidx[r, pl.ds(c0, 16)] = v + v + ci
      return 0
    lax.fori_loop(0, NB * (B // 16), dbl, 0)

    table = y2

    def start_gather(b, buf, sem):
      pltpu.make_async_copy(table.at[sidx.at[b]], buf, sem).start()

    def wait_gather(b, buf, sem):
      pltpu.make_async_copy(table.at[sidx.at[b]], buf, sem).wait()

    def scatter(b, buf, parity):
      pltpu.sync_copy(buf, acc.at[didx.at[b]], add=True)
      if with_counts:
        @pl.when(ci == parity)
        def _():
          pltpu.sync_copy(ones_v, cnt.at[didx.at[b]], add=True)

    slots = ((buf_b, sem_b, 0), (buf_c, sem_c, 1),
             (buf_d, sem_d, 0), (buf_a, sem_a, 1))

    for j in range(3):
      start_gather(j, slots[j][0], slots[j][1])

    def zf(i, _):
      buf_a[i // (H // 16), pl.ds((i % (H // 16)) * 16, 16)] = (
          jnp.zeros((16,), jnp.float32))
      return 0
    lax.fori_loop(0, B * H // 16, zf, 0)
    r0 = si * ROWS_PER_TILE
    for k in range(ROWS_PER_TILE // B):
      pltpu.sync_copy(buf_a, acc.at[pl.ds(r0 + k * B, B)])
    rem = ROWS_PER_TILE % B
    if rem:
      pltpu.sync_copy(buf_a.at[pl.ds(0, rem)],
                      acc.at[pl.ds(r0 + (ROWS_PER_TILE // B) * B, rem)])

    if with_counts:
      def of(i, _):
        ones_v[i, pl.ds(0, 16)] = jnp.ones((16,), jnp.float32)
        zer16[i, pl.ds(0, 16)] = jnp.zeros((16,), jnp.float32)
        return 0
      lax.fori_loop(0, B, of, 0)
      for k in range(ROWS_PER_TILE // B):
        pltpu.sync_copy(zer16, cnt.at[pl.ds(r0 + k * B, B)])
      if rem:
        pltpu.sync_copy(zer16.at[pl.ds(0, rem)],
                        cnt.at[pl.ds(r0 + (ROWS_PER_TILE // B) * B, rem)])

    plsc.subcore_barrier()

    def it(i, _):
      b = 4 * i
      start_gather(b + 3, buf_a, sem_a)
      for j, (buf, sem, parity) in enumerate(slots):
        wait_gather(b + j, buf, sem)
        scatter(b + j, buf, parity)
        if j < 3:
          @pl.when(i < NIT - 1)
          def _():
            start_gather(b + 4 + j, buf, sem)
      return 0

    lax.fori_loop(0, NIT, it, 0)
    plsc.subcore_barrier()

    pltpu.sync_copy(acc.at[pl.ds(r0, ROWS_PER_TILE)],
                    p_out.at[pl.ds(r0, ROWS_PER_TILE), pl.ds(ci * H, H)])
    if with_counts:
      pltpu.sync_copy(cnt.at[pl.ds(r0, ROWS_PER_TILE)],
                      c_out.at[pl.ds(r0, ROWS_PER_TILE), pl.ds(ci * 16, 16)])

  return pl.kernel(
      body, out_type=out_type, mesh=mesh, scratch_types=scratch,
      compiler_params=pltpu.CompilerParams(use_tc_tiling_on_sc=False))


_agg_with_counts = _make_agg(True)
_agg_plain = _make_agg(False)

BM = 2000


def _pre(x, W_l):
  def body(x_ref, w_ref, o_ref):
    o_ref[...] = lax.dot_general(x_ref[...], w_ref[...],
                                 (((1,), (1,)), ((), ())),
                                 preferred_element_type=jnp.float32)
  return pl.pallas_call(
      body,
      grid=(N // BM,),
      in_specs=[pl.BlockSpec((BM, D), lambda i: (i, 0)),
                pl.BlockSpec((D, D), lambda i: (0, 0))],
      out_specs=pl.BlockSpec((BM, D), lambda i: (i, 0)),
      out_shape=jax.ShapeDtypeStruct((N, D), jnp.float32),
  )(x, W_l)


def _unpack_mean(p_ref, c_ref):
  cnt = c_ref[:, 0:1] + c_ref[:, 16:17]
  return p_ref[...] * (1.0 / jnp.maximum(cnt, 1.0))


def _post(p, cnts, b, x, W_r, relu):
  def body(p_ref, c_ref, b_ref, x_ref, w_ref, o_ref):
    r = _unpack_mean(p_ref, c_ref) + b_ref[...] + lax.dot_general(
        x_ref[...], w_ref[...], (((1,), (1,)), ((), ())),
        preferred_element_type=jnp.float32)
    o_ref[...] = jnp.maximum(r, 0.0) if relu else r
  return pl.pallas_call(
      body,
      grid=(N // BM,),
      in_specs=[
          pl.BlockSpec((BM, D), lambda i: (i, 0)),
          pl.BlockSpec((BM, D), lambda i: (i, 0)),
          pl.BlockSpec((1, D), lambda i: (0, 0)),
          pl.BlockSpec((BM, D), lambda i: (i, 0)),
          pl.BlockSpec((D, D), lambda i: (0, 0)),
      ],
      out_specs=pl.BlockSpec((BM, D), lambda i: (i, 0)),
      out_shape=jax.ShapeDtypeStruct((N, D), jnp.float32),
  )(p, cnts, b, x, W_r)


def _post_pre(p, cnts, b, x, W_r, W_ln):
  def body(p_ref, c_ref, b_ref, x_ref, w_ref, wl_ref, h_ref, y_ref):
    r = _unpack_mean(p_ref, c_ref) + b_ref[...] + lax.dot_general(
        x_ref[...], w_ref[...], (((1,), (1,)), ((), ())),
        preferred_element_type=jnp.float32)
    h = jnp.maximum(r, 0.0)
    h_ref[...] = h
    y_ref[...] = lax.dot_general(h, wl_ref[...], (((1,), (1,)), ((), ())),
                                 preferred_element_type=jnp.float32)
  return pl.pallas_call(
      body,
      grid=(N // BM,),
      in_specs=[
          pl.BlockSpec((BM, D), lambda i: (i, 0)),
          pl.BlockSpec((BM, D), lambda i: (i, 0)),
          pl.BlockSpec((1, D), lambda i: (0, 0)),
          pl.BlockSpec((BM, D), lambda i: (i, 0)),
          pl.BlockSpec((D, D), lambda i: (0, 0)),
          pl.BlockSpec((D, D), lambda i: (0, 0)),
      ],
      out_specs=[pl.BlockSpec((BM, D), lambda i: (i, 0)),
                 pl.BlockSpec((BM, D), lambda i: (i, 0))],
      out_shape=[jax.ShapeDtypeStruct((N, D), jnp.float32),
                 jax.ShapeDtypeStruct((N, D), jnp.float32)],
  )(p, cnts, b, x, W_r, W_ln)


def kernel(x, edge_index, W1_l, b1, W1_r, W2_l, b2, W2_r):
  ei = edge_index.astype(jnp.int32)
  pad_i = np.arange(PAD, dtype=np.int32)
  pad_src = jnp.asarray(((pad_i * 37) % N).reshape(PAD // B, B))
  pad_dst = jnp.asarray((N + pad_i % (N_PAD - N)).reshape(PAD // B, B))
  src2 = jnp.concatenate([ei[0].reshape(E // B, B), pad_src], axis=0)
  dst2 = jnp.concatenate([ei[1].reshape(E // B, B), pad_dst], axis=0)
  src2, dst2 = lax.optimization_barrier((src2, dst2))
  b1r = b1.reshape(1, D)
  b2r = b2.reshape(1, D)

  y1 = jnp.reshape(_pre(x, W1_l), (2 * N, H))
  p1, cnts = _agg_with_counts(y1, src2, dst2)
  h, y2 = _post_pre(p1, cnts, b1r, x, W1_r, W2_l)
  (p2,) = _agg_plain(jnp.reshape(y2, (2 * N, H)), src2, dst2)
  out = _post(p2, cnts, b2r, h, W2_r, relu=False)
  return out
